# baseline (speedup 1.0000x reference)
import jax
import jax.numpy as jnp
from jax.experimental import pallas as pl

_N = 100000
_D = 128
_BR = 2000


def _logic_kernel(x_ref, c_ref, wt_ref, b_ref, coef_ref, o_ref):
    x = x_ref[...]
    c = c_ref[...]
    t = jnp.dot(x, wt_ref[...], preferred_element_type=jnp.float32)
    t = jnp.maximum(t + b_ref[...], 0.0)
    a = coef_ref[0:1, :]
    m = coef_ref[1:2, :]
    o_ref[...] = t + a * jnp.minimum(x, c) + m * jnp.maximum(x, c)


def kernel(x, context, W, b, logical_weight, neural_weight):
    lw = jax.nn.sigmoid(logical_weight)
    nw = jax.nn.sigmoid(neural_weight)
    wt = (nw * W).T
    b2 = (nw * b).reshape(1, _D)
    coef = jnp.stack([
        jnp.full((_D,), (1.0 - nw) * lw, dtype=jnp.float32),
        jnp.full((_D,), (1.0 - nw) * (1.0 - lw), dtype=jnp.float32),
    ])
    grid = _N // _BR
    return pl.pallas_call(
        _logic_kernel,
        grid=(grid,),
        in_specs=[
            pl.BlockSpec((_BR, _D), lambda i: (i, 0)),
            pl.BlockSpec((_BR, _D), lambda i: (i, 0)),
            pl.BlockSpec((_D, _D), lambda i: (0, 0)),
            pl.BlockSpec((1, _D), lambda i: (0, 0)),
            pl.BlockSpec((2, _D), lambda i: (0, 0)),
        ],
        out_specs=pl.BlockSpec((_BR, _D), lambda i: (i, 0)),
        out_shape=jax.ShapeDtypeStruct((_N, _D), jnp.float32),
    )(x, context, wt, b2, coef)


# speedup vs baseline: 1.0000x; 1.0000x over previous
import jax
import jax.numpy as jnp
from jax.experimental import pallas as pl
from jax.experimental.pallas import tpu as pltpu

_N = 100000
_D = 128
_BR = 2000


def _logic_kernel(x_ref, c_ref, wt_ref, b_ref, coef_ref, o_ref):
    x = x_ref[...]
    c = c_ref[...]
    t = jnp.dot(x, wt_ref[...], preferred_element_type=jnp.float32)
    t = jnp.maximum(t + b_ref[...], 0.0)
    a = coef_ref[0:1, :]
    m = coef_ref[1:2, :]
    o_ref[...] = t + a * jnp.minimum(x, c) + m * jnp.maximum(x, c)


def kernel(x, context, W, b, logical_weight, neural_weight):
    lw = jax.nn.sigmoid(logical_weight)
    nw = jax.nn.sigmoid(neural_weight)
    wt = (nw * W).T
    b2 = (nw * b).reshape(1, _D)
    coef = jnp.stack([
        jnp.full((_D,), (1.0 - nw) * lw, dtype=jnp.float32),
        jnp.full((_D,), (1.0 - nw) * (1.0 - lw), dtype=jnp.float32),
    ])
    grid = _N // _BR
    return pl.pallas_call(
        _logic_kernel,
        grid=(grid,),
        in_specs=[
            pl.BlockSpec((_BR, _D), lambda i: (i, 0)),
            pl.BlockSpec((_BR, _D), lambda i: (i, 0)),
            pl.BlockSpec((_D, _D), lambda i: (0, 0)),
            pl.BlockSpec((1, _D), lambda i: (0, 0)),
            pl.BlockSpec((2, _D), lambda i: (0, 0)),
        ],
        out_specs=pl.BlockSpec((_BR, _D), lambda i: (i, 0)),
        out_shape=jax.ShapeDtypeStruct((_N, _D), jnp.float32),
        compiler_params=pltpu.CompilerParams(
            dimension_semantics=("parallel",),
        ),
    )(x, context, wt, b2, coef)


# speedup vs baseline: 1.2480x; 1.2480x over previous
import jax
import jax.numpy as jnp
from jax.experimental import pallas as pl
from jax.experimental.pallas import tpu as pltpu

_N = 100000
_D = 128
_BR = 4000


def _logic_kernel(x_ref, c_ref, wt_ref, b_ref, coef_ref, o_ref):
    x = x_ref[...]
    c = c_ref[...]
    t = jnp.dot(x, wt_ref[...], preferred_element_type=jnp.float32)
    t = jnp.maximum(t + b_ref[...], 0.0)
    a = coef_ref[0:1, :]
    m = coef_ref[1:2, :]
    o_ref[...] = t + a * jnp.minimum(x, c) + m * jnp.maximum(x, c)


def kernel(x, context, W, b, logical_weight, neural_weight):
    lw = jax.nn.sigmoid(logical_weight)
    nw = jax.nn.sigmoid(neural_weight)
    wt = (nw * W).T
    b2 = (nw * b).reshape(1, _D)
    coef = jnp.stack([
        jnp.full((_D,), (1.0 - nw) * lw, dtype=jnp.float32),
        jnp.full((_D,), (1.0 - nw) * (1.0 - lw), dtype=jnp.float32),
    ])
    grid = _N // _BR
    return pl.pallas_call(
        _logic_kernel,
        grid=(grid,),
        in_specs=[
            pl.BlockSpec((_BR, _D), lambda i: (i, 0)),
            pl.BlockSpec((_BR, _D), lambda i: (i, 0)),
            pl.BlockSpec((_D, _D), lambda i: (0, 0)),
            pl.BlockSpec((1, _D), lambda i: (0, 0)),
            pl.BlockSpec((2, _D), lambda i: (0, 0)),
        ],
        out_specs=pl.BlockSpec((_BR, _D), lambda i: (i, 0)),
        out_shape=jax.ShapeDtypeStruct((_N, _D), jnp.float32),
        compiler_params=pltpu.CompilerParams(
            dimension_semantics=("parallel",),
        ),
    )(x, context, wt, b2, coef)


# speedup vs baseline: 1.3370x; 1.0713x over previous
import jax
import jax.numpy as jnp
from jax.experimental import pallas as pl
from jax.experimental.pallas import tpu as pltpu

_N = 100000
_D = 128
_BR = 10000


def _logic_kernel(x_ref, c_ref, wt_ref, b_ref, coef_ref, o_ref):
    x = x_ref[...]
    c = c_ref[...]
    t = jnp.dot(x, wt_ref[...], preferred_element_type=jnp.float32)
    t = jnp.maximum(t + b_ref[...], 0.0)
    a = coef_ref[0:1, :]
    m = coef_ref[1:2, :]
    o_ref[...] = t + a * jnp.minimum(x, c) + m * jnp.maximum(x, c)


def kernel(x, context, W, b, logical_weight, neural_weight):
    lw = jax.nn.sigmoid(logical_weight)
    nw = jax.nn.sigmoid(neural_weight)
    wt = (nw * W).T
    b2 = (nw * b).reshape(1, _D)
    coef = jnp.stack([
        jnp.full((_D,), (1.0 - nw) * lw, dtype=jnp.float32),
        jnp.full((_D,), (1.0 - nw) * (1.0 - lw), dtype=jnp.float32),
    ])
    grid = _N // _BR
    return pl.pallas_call(
        _logic_kernel,
        grid=(grid,),
        in_specs=[
            pl.BlockSpec((_BR, _D), lambda i: (i, 0)),
            pl.BlockSpec((_BR, _D), lambda i: (i, 0)),
            pl.BlockSpec((_D, _D), lambda i: (0, 0)),
            pl.BlockSpec((1, _D), lambda i: (0, 0)),
            pl.BlockSpec((2, _D), lambda i: (0, 0)),
        ],
        out_specs=pl.BlockSpec((_BR, _D), lambda i: (i, 0)),
        out_shape=jax.ShapeDtypeStruct((_N, _D), jnp.float32),
        compiler_params=pltpu.CompilerParams(
            dimension_semantics=("parallel",),
        ),
    )(x, context, wt, b2, coef)


# speedup vs baseline: 1.3935x; 1.0422x over previous
import jax
import jax.numpy as jnp
from jax.experimental import pallas as pl
from jax.experimental.pallas import tpu as pltpu

_N = 100000
_D = 128
_BR = 16000


def _logic_kernel(x_ref, c_ref, wt_ref, b_ref, coef_ref, o_ref):
    x = x_ref[...]
    c = c_ref[...]
    t = jnp.dot(x, wt_ref[...], preferred_element_type=jnp.float32)
    t = jnp.maximum(t + b_ref[...], 0.0)
    a = coef_ref[0:1, :]
    m = coef_ref[1:2, :]
    o_ref[...] = t + a * jnp.minimum(x, c) + m * jnp.maximum(x, c)


def kernel(x, context, W, b, logical_weight, neural_weight):
    lw = jax.nn.sigmoid(logical_weight)
    nw = jax.nn.sigmoid(neural_weight)
    wt = (nw * W).T
    b2 = (nw * b).reshape(1, _D)
    coef = jnp.stack([
        jnp.full((_D,), (1.0 - nw) * lw, dtype=jnp.float32),
        jnp.full((_D,), (1.0 - nw) * (1.0 - lw), dtype=jnp.float32),
    ])
    grid = (_N + _BR - 1) // _BR
    return pl.pallas_call(
        _logic_kernel,
        grid=(grid,),
        in_specs=[
            pl.BlockSpec((_BR, _D), lambda i: (i, 0)),
            pl.BlockSpec((_BR, _D), lambda i: (i, 0)),
            pl.BlockSpec((_D, _D), lambda i: (0, 0)),
            pl.BlockSpec((1, _D), lambda i: (0, 0)),
            pl.BlockSpec((2, _D), lambda i: (0, 0)),
        ],
        out_specs=pl.BlockSpec((_BR, _D), lambda i: (i, 0)),
        out_shape=jax.ShapeDtypeStruct((_N, _D), jnp.float32),
        compiler_params=pltpu.CompilerParams(
            dimension_semantics=("parallel",),
        ),
    )(x, context, wt, b2, coef)
